# Initial kernel scaffold; baseline (speedup 1.0000x reference)
import jax
import jax.numpy as jnp
from jax.experimental import pallas as pl


def kernel(feat_lr, guide_hr, sx_raw, sy_raw, th_raw, sr_raw):
    raise NotImplementedError("write your pallas kernel here")


# speedup vs baseline: 97.6701x; 97.6701x over previous
import functools
import math

import jax
import jax.numpy as jnp
from jax import lax
from jax.experimental import pallas as pl
from jax.experimental.pallas import tpu as pltpu
from jax.experimental.pallas import tpu_sc as plsc

_SCALE = 16
_RMAX = 3
_HL = 14
_WL = 14
_C = 64
_NT = _HL * _WL
_OFFS = tuple((dy, dx) for dy in range(-_RMAX, _RMAX + 1)
              for dx in range(-_RMAX, _RMAX + 1)
              if dy * dy + dx * dx <= _RMAX * _RMAX)
_K = len(_OFFS)
_NW = 32
_TPW = -(-_NT // _NW)
_NEG = -1e30


def _bilinear_resize(img, Ho, Wo):
    B, C, Hi, Wi = img.shape
    sy = Hi / Ho
    sx = Wi / Wo
    ys = jnp.maximum((jnp.arange(Ho, dtype=jnp.float32) + 0.5) * sy - 0.5, 0.0)
    xs = jnp.maximum((jnp.arange(Wo, dtype=jnp.float32) + 0.5) * sx - 0.5, 0.0)
    y0 = jnp.clip(jnp.floor(ys).astype(jnp.int32), 0, Hi - 1)
    x0 = jnp.clip(jnp.floor(xs).astype(jnp.int32), 0, Wi - 1)
    y1 = jnp.minimum(y0 + 1, Hi - 1)
    x1 = jnp.minimum(x0 + 1, Wi - 1)
    wy = (ys - y0.astype(jnp.float32))[:, None]
    wx = (xs - x0.astype(jnp.float32))[None, :]
    v00 = img[:, :, y0[:, None], x0[None, :]]
    v01 = img[:, :, y0[:, None], x1[None, :]]
    v10 = img[:, :, y1[:, None], x0[None, :]]
    v11 = img[:, :, y1[:, None], x1[None, :]]
    top = (1.0 - wx) * v00 + wx * v01
    bot = (1.0 - wx) * v10 + wx * v11
    return (1.0 - wy) * top + wy * bot


def _splat_i32(v):
    return jnp.full((16,), v, jnp.int32)


def _jbu_sc_body(featv_h, pv_h, ght_h, r2_h, out_h,
                 featv, pv, ght, r2v, wbuf, mbuf, dbuf, acc):
    wid = lax.axis_index("s") * 2 + lax.axis_index("c")
    pltpu.sync_copy(featv_h, featv)
    pltpu.sync_copy(pv_h, pv)
    col = [_splat_i32(c) for c in range(8)]
    xio = lax.iota(jnp.int32, 16).astype(jnp.float32)

    def do_tile(tile):
        i = tile // _WL
        j = tile - i * _WL
        pltpu.sync_copy(ght_h.at[tile], ght)
        pltpu.sync_copy(r2_h.at[tile], r2v)
        yb = (i * 16).astype(jnp.float32)
        xb = (j * 16).astype(jnp.float32)

        for k, (dy, dx) in enumerate(_OFFS):
            ni = jnp.clip(i + dy, 0, _HL - 1)
            nj = jnp.clip(j + dx, 0, _WL - 1)
            f = ni * _WL + nj
            fsp = _splat_i32(f)
            a = plsc.load_gather(pv, [fsp, col[0]])
            b = plsc.load_gather(pv, [fsp, col[1]])
            ct = plsc.load_gather(pv, [fsp, col[2]])
            st = plsc.load_gather(pv, [fsp, col[3]])
            cr = plsc.load_gather(pv, [fsp, col[4]])
            g0 = plsc.load_gather(pv, [fsp, col[5]])
            g1 = plsc.load_gather(pv, [fsp, col[6]])
            g2 = plsc.load_gather(pv, [fsp, col[7]])
            cxs = (nj.astype(jnp.float32) + 0.5) * float(_SCALE) - 0.5
            cys = (ni.astype(jnp.float32) + 0.5) * float(_SCALE) - 0.5
            dxv = xio + (xb - cxs)
            rad2 = float(dy * dy + dx * dx)

            def row1(r, _, k=k, dxv=dxv, a=a, b=b, ct=ct, st=st, cr=cr,
                     g0=g0, g1=g1, g2=g2, cys=cys):
                p = r * 16
                dyv = yb + r.astype(jnp.float32) - cys
                xp = dxv * ct + dyv * st
                yp = dyv * ct - dxv * st
                gh0 = ght[pl.ds(p, 16)]
                gh1 = ght[pl.ds(256 + p, 16)]
                gh2 = ght[pl.ds(512 + p, 16)]
                e0 = gh0 - g0
                e1 = gh1 - g1
                e2 = gh2 - g2
                d2 = e0 * e0 + e1 * e1 + e2 * e2
                lw = -(xp * xp * a + yp * yp * b) - d2 * cr
                mask = r2v[pl.ds(p, 16)] >= rad2
                lwm = jnp.where(mask, lw, _NEG)
                wbuf[k, pl.ds(p, 16)] = lwm
                if k == 0:
                    mbuf[pl.ds(p, 16)] = lwm
                else:
                    mbuf[pl.ds(p, 16)] = jnp.maximum(mbuf[pl.ds(p, 16)], lwm)
                return 0

            lax.fori_loop(0, 16, row1, 0, unroll=False)

        def row2(r, _):
            p = r * 16
            m = mbuf[pl.ds(p, 16)]
            den = jnp.zeros((16,), jnp.float32)
            for k in range(_K):
                s = jnp.exp(wbuf[k, pl.ds(p, 16)] - m)
                wbuf[k, pl.ds(p, 16)] = s
                den = den + s
            dbuf[pl.ds(p, 16)] = 1.0 / den
            return 0

        lax.fori_loop(0, 16, row2, 0, unroll=False)

        def cblk(cb, _):
            cols = [_splat_i32(cb * 8 + co) for co in range(8)]

            def rblk(rb, _, cols=cols, cb=cb):
                p0 = rb * 64
                sv = [[wbuf[k, pl.ds(p0 + rr * 16, 16)] for rr in range(4)]
                      for k in range(_K)]
                accs = [[jnp.zeros((16,), jnp.float32) for _ in range(4)]
                        for _ in range(8)]
                for k, (dy, dx) in enumerate(_OFFS):
                    ni = jnp.clip(i + dy, 0, _HL - 1)
                    nj = jnp.clip(j + dx, 0, _WL - 1)
                    fsp = _splat_i32(ni * _WL + nj)
                    for co in range(8):
                        fv = plsc.load_gather(featv, [fsp, cols[co]])
                        for rr in range(4):
                            accs[co][rr] = accs[co][rr] + fv * sv[k][rr]
                for rr in range(4):
                    inv = dbuf[pl.ds(p0 + rr * 16, 16)]
                    for co in range(8):
                        acc[cb * 8 + co, pl.ds(p0 + rr * 16, 16)] = \
                            accs[co][rr] * inv
                return 0

            lax.fori_loop(0, 4, rblk, 0, unroll=False)
            return 0

        lax.fori_loop(0, 8, cblk, 0, unroll=False)
        pltpu.sync_copy(acc, out_h.at[tile])

    def tloop(t, _):
        tile = wid + _NW * t

        @pl.when(tile < _NT)
        def _():
            do_tile(tile)

        return 0

    lax.fori_loop(0, _TPW, tloop, 0, unroll=False)


@jax.jit
def kernel(feat_lr, guide_hr, sx_raw, sy_raw, th_raw, sr_raw):
    B, C, Hl, Wl = feat_lr.shape
    _, _, Hh, Wh = guide_hr.shape
    sigma_x = jnp.exp(sx_raw)
    sigma_y = jnp.exp(sy_raw)
    theta = math.pi * jnp.tanh(th_raw)
    sigma_r = jnp.exp(sr_raw)
    sx = jnp.maximum(sigma_x, 1e-6)[0, 0]
    sy = jnp.maximum(sigma_y, 1e-6)[0, 0]
    sr = jnp.maximum(sigma_r, 1e-6)[0, 0]
    a_m = 1.0 / (2.0 * sx * sx + 1e-8)
    b_m = 1.0 / (2.0 * sy * sy + 1e-8)
    cr_m = 1.0 / (2.0 * sr * sr + 1e-8)
    cos_m = jnp.cos(theta[0, 0])
    sin_m = jnp.sin(theta[0, 0])
    glr = _bilinear_resize(guide_hr, Hl, Wl)[0]
    pv = jnp.zeros((Hl, Wl, 16), jnp.float32)
    for cidx, m in enumerate([a_m, b_m, cos_m, sin_m, cr_m,
                              glr[0], glr[1], glr[2]]):
        pv = pv.at[:, :, cidx].set(m)
    pv = pv.reshape(_NT, 16)
    sigma_eff = jnp.maximum(sigma_x, sigma_y)
    sigma_eff_hr = _bilinear_resize(sigma_eff, Hh, Wh)[0, 0]
    R_map = jnp.clip(jnp.ceil(2.0 * sigma_eff_hr), 1, _RMAX)
    r2 = (R_map * R_map).astype(jnp.float32)
    r2t = r2.reshape(Hl, 16, Wl, 16).transpose(0, 2, 1, 3).reshape(_NT, 256)
    featv = feat_lr[0].transpose(1, 2, 0).reshape(_NT, _C)
    ght = (guide_hr[0].reshape(3, Hl, 16, Wl, 16)
           .transpose(1, 3, 0, 2, 4).reshape(_NT, 3 * 256))

    mesh = plsc.VectorSubcoreMesh(core_axis_name="c", subcore_axis_name="s",
                                  num_cores=2, num_subcores=16)
    out_t = pl.kernel(
        _jbu_sc_body,
        mesh=mesh,
        compiler_params=pltpu.CompilerParams(needs_layout_passes=False),
        out_type=jax.ShapeDtypeStruct((_NT, _C, 256), jnp.float32),
        scratch_types=[
            pltpu.VMEM((_NT, _C), jnp.float32),
            pltpu.VMEM((_NT, 16), jnp.float32),
            pltpu.VMEM((3 * 256,), jnp.float32),
            pltpu.VMEM((256,), jnp.float32),
            pltpu.VMEM((_K, 256), jnp.float32),
            pltpu.VMEM((256,), jnp.float32),
            pltpu.VMEM((256,), jnp.float32),
            pltpu.VMEM((_C, 256), jnp.float32),
        ],
    )(featv, pv, ght, r2t)
    out = (out_t.reshape(Hl, Wl, _C, 16, 16)
           .transpose(2, 0, 3, 1, 4).reshape(1, _C, Hh, Wh))
    return out


# speedup vs baseline: 97.8987x; 1.0023x over previous
import functools
import math

import jax
import jax.numpy as jnp
from jax import lax
from jax.experimental import pallas as pl
from jax.experimental.pallas import tpu as pltpu
from jax.experimental.pallas import tpu_sc as plsc

_SCALE = 16
_RMAX = 3
_HL = 14
_WL = 14
_C = 64
_NT = _HL * _WL
_OFFS = tuple((dy, dx) for dy in range(-_RMAX, _RMAX + 1)
              for dx in range(-_RMAX, _RMAX + 1)
              if dy * dy + dx * dx <= _RMAX * _RMAX)
_K = len(_OFFS)
_NW = 32
_TPW = -(-_NT // _NW)
_NEG = -1e30


def _bilinear_resize(img, Ho, Wo):
    B, C, Hi, Wi = img.shape
    sy = Hi / Ho
    sx = Wi / Wo
    ys = jnp.maximum((jnp.arange(Ho, dtype=jnp.float32) + 0.5) * sy - 0.5, 0.0)
    xs = jnp.maximum((jnp.arange(Wo, dtype=jnp.float32) + 0.5) * sx - 0.5, 0.0)
    y0 = jnp.clip(jnp.floor(ys).astype(jnp.int32), 0, Hi - 1)
    x0 = jnp.clip(jnp.floor(xs).astype(jnp.int32), 0, Wi - 1)
    y1 = jnp.minimum(y0 + 1, Hi - 1)
    x1 = jnp.minimum(x0 + 1, Wi - 1)
    wy = (ys - y0.astype(jnp.float32))[:, None]
    wx = (xs - x0.astype(jnp.float32))[None, :]
    v00 = img[:, :, y0[:, None], x0[None, :]]
    v01 = img[:, :, y0[:, None], x1[None, :]]
    v10 = img[:, :, y1[:, None], x0[None, :]]
    v11 = img[:, :, y1[:, None], x1[None, :]]
    top = (1.0 - wx) * v00 + wx * v01
    bot = (1.0 - wx) * v10 + wx * v11
    return (1.0 - wy) * top + wy * bot


def _splat_i32(v):
    return jnp.full((16,), v, jnp.int32)


def _jbu_sc_body(featv_h, pv_h, ght_h, r2_h, out_h,
                 featv, pv, ght, r2v, wbuf, mbuf, dbuf, acc):
    wid = lax.axis_index("s") * 2 + lax.axis_index("c")
    pltpu.sync_copy(featv_h, featv)
    pltpu.sync_copy(pv_h, pv)
    col = [_splat_i32(c) for c in range(8)]
    xio = lax.iota(jnp.int32, 16).astype(jnp.float32)

    def do_tile(tile):
        i = tile // _WL
        j = tile - i * _WL
        pltpu.sync_copy(ght_h.at[tile], ght)
        pltpu.sync_copy(r2_h.at[tile], r2v)
        yb = (i * 16).astype(jnp.float32)
        xb = (j * 16).astype(jnp.float32)

        for k, (dy, dx) in enumerate(_OFFS):
            ni = jnp.clip(i + dy, 0, _HL - 1)
            nj = jnp.clip(j + dx, 0, _WL - 1)
            f = ni * _WL + nj
            fsp = _splat_i32(f)
            a = plsc.load_gather(pv, [fsp, col[0]])
            b = plsc.load_gather(pv, [fsp, col[1]])
            ct = plsc.load_gather(pv, [fsp, col[2]])
            st = plsc.load_gather(pv, [fsp, col[3]])
            cr = plsc.load_gather(pv, [fsp, col[4]])
            g0 = plsc.load_gather(pv, [fsp, col[5]])
            g1 = plsc.load_gather(pv, [fsp, col[6]])
            g2 = plsc.load_gather(pv, [fsp, col[7]])
            cxs = (nj.astype(jnp.float32) + 0.5) * float(_SCALE) - 0.5
            cys = (ni.astype(jnp.float32) + 0.5) * float(_SCALE) - 0.5
            dxv = xio + (xb - cxs)
            rad2 = float(dy * dy + dx * dx)

            def row1(r, _, k=k, dxv=dxv, a=a, b=b, ct=ct, st=st, cr=cr,
                     g0=g0, g1=g1, g2=g2, cys=cys):
                p = r * 16
                dyv = yb + r.astype(jnp.float32) - cys
                xp = dxv * ct + dyv * st
                yp = dyv * ct - dxv * st
                gh0 = ght[pl.ds(p, 16)]
                gh1 = ght[pl.ds(256 + p, 16)]
                gh2 = ght[pl.ds(512 + p, 16)]
                e0 = gh0 - g0
                e1 = gh1 - g1
                e2 = gh2 - g2
                d2 = e0 * e0 + e1 * e1 + e2 * e2
                lw = -(xp * xp * a + yp * yp * b) - d2 * cr
                mask = r2v[pl.ds(p, 16)] >= rad2
                lwm = jnp.where(mask, lw, _NEG)
                wbuf[k, pl.ds(p, 16)] = lwm
                if k == 0:
                    mbuf[pl.ds(p, 16)] = lwm
                else:
                    mbuf[pl.ds(p, 16)] = jnp.maximum(mbuf[pl.ds(p, 16)], lwm)
                return 0

            lax.fori_loop(0, 16, row1, 0, unroll=False)

        def row2(r, _):
            p = r * 16
            m = mbuf[pl.ds(p, 16)]
            den = jnp.zeros((16,), jnp.float32)
            for k in range(_K):
                s = jnp.exp(wbuf[k, pl.ds(p, 16)] - m)
                wbuf[k, pl.ds(p, 16)] = s
                den = den + s
            dbuf[pl.ds(p, 16)] = 1.0 / den
            return 0

        lax.fori_loop(0, 16, row2, 0, unroll=False)

        def cblk(cb, _):
            cols = [_splat_i32(cb * 8 + co) for co in range(8)]

            def rblk(rb, _, cols=cols, cb=cb):
                p0 = rb * 64
                accs = [[jnp.zeros((16,), jnp.float32) for _ in range(4)]
                        for _ in range(8)]
                for k, (dy, dx) in enumerate(_OFFS):
                    ni = jnp.clip(i + dy, 0, _HL - 1)
                    nj = jnp.clip(j + dx, 0, _WL - 1)
                    fsp = _splat_i32(ni * _WL + nj)
                    sv = [wbuf[k, pl.ds(p0 + rr * 16, 16)] for rr in range(4)]
                    for co in range(8):
                        fv = plsc.load_gather(featv, [fsp, cols[co]])
                        for rr in range(4):
                            accs[co][rr] = accs[co][rr] + fv * sv[rr]
                for rr in range(4):
                    inv = dbuf[pl.ds(p0 + rr * 16, 16)]
                    for co in range(8):
                        acc[cb * 8 + co, pl.ds(p0 + rr * 16, 16)] = \
                            accs[co][rr] * inv
                return 0

            lax.fori_loop(0, 4, rblk, 0, unroll=False)
            return 0

        lax.fori_loop(0, 8, cblk, 0, unroll=False)
        pltpu.sync_copy(acc, out_h.at[tile])

    def tloop(t, _):
        tile = wid + _NW * t

        @pl.when(tile < _NT)
        def _():
            do_tile(tile)

        return 0

    lax.fori_loop(0, _TPW, tloop, 0, unroll=False)


@jax.jit
def kernel(feat_lr, guide_hr, sx_raw, sy_raw, th_raw, sr_raw):
    B, C, Hl, Wl = feat_lr.shape
    _, _, Hh, Wh = guide_hr.shape
    sigma_x = jnp.exp(sx_raw)
    sigma_y = jnp.exp(sy_raw)
    theta = math.pi * jnp.tanh(th_raw)
    sigma_r = jnp.exp(sr_raw)
    sx = jnp.maximum(sigma_x, 1e-6)[0, 0]
    sy = jnp.maximum(sigma_y, 1e-6)[0, 0]
    sr = jnp.maximum(sigma_r, 1e-6)[0, 0]
    a_m = 1.0 / (2.0 * sx * sx + 1e-8)
    b_m = 1.0 / (2.0 * sy * sy + 1e-8)
    cr_m = 1.0 / (2.0 * sr * sr + 1e-8)
    cos_m = jnp.cos(theta[0, 0])
    sin_m = jnp.sin(theta[0, 0])
    glr = _bilinear_resize(guide_hr, Hl, Wl)[0]
    pv = jnp.zeros((Hl, Wl, 16), jnp.float32)
    for cidx, m in enumerate([a_m, b_m, cos_m, sin_m, cr_m,
                              glr[0], glr[1], glr[2]]):
        pv = pv.at[:, :, cidx].set(m)
    pv = pv.reshape(_NT, 16)
    sigma_eff = jnp.maximum(sigma_x, sigma_y)
    sigma_eff_hr = _bilinear_resize(sigma_eff, Hh, Wh)[0, 0]
    R_map = jnp.clip(jnp.ceil(2.0 * sigma_eff_hr), 1, _RMAX)
    r2 = (R_map * R_map).astype(jnp.float32)
    r2t = r2.reshape(Hl, 16, Wl, 16).transpose(0, 2, 1, 3).reshape(_NT, 256)
    featv = feat_lr[0].transpose(1, 2, 0).reshape(_NT, _C)
    ght = (guide_hr[0].reshape(3, Hl, 16, Wl, 16)
           .transpose(1, 3, 0, 2, 4).reshape(_NT, 3 * 256))

    mesh = plsc.VectorSubcoreMesh(core_axis_name="c", subcore_axis_name="s",
                                  num_cores=2, num_subcores=16)
    out_t = pl.kernel(
        _jbu_sc_body,
        mesh=mesh,
        compiler_params=pltpu.CompilerParams(needs_layout_passes=False),
        out_type=jax.ShapeDtypeStruct((_NT, _C, 256), jnp.float32),
        scratch_types=[
            pltpu.VMEM((_NT, _C), jnp.float32),
            pltpu.VMEM((_NT, 16), jnp.float32),
            pltpu.VMEM((3 * 256,), jnp.float32),
            pltpu.VMEM((256,), jnp.float32),
            pltpu.VMEM((_K, 256), jnp.float32),
            pltpu.VMEM((256,), jnp.float32),
            pltpu.VMEM((256,), jnp.float32),
            pltpu.VMEM((_C, 256), jnp.float32),
        ],
    )(featv, pv, ght, r2t)
    out = (out_t.reshape(Hl, Wl, _C, 16, 16)
           .transpose(2, 0, 3, 1, 4).reshape(1, _C, Hh, Wh))
    return out


# speedup vs baseline: 711.3448x; 7.2661x over previous
import functools
import math

import jax
import jax.numpy as jnp
from jax import lax
from jax.experimental import pallas as pl
from jax.experimental.pallas import tpu as pltpu
from jax.experimental.pallas import tpu_sc as plsc

_SCALE = 16
_RMAX = 3
_HL = 14
_WL = 14
_C = 64
_NT = _HL * _WL
_OFFS = tuple((dy, dx) for dy in range(-_RMAX, _RMAX + 1)
              for dx in range(-_RMAX, _RMAX + 1)
              if dy * dy + dx * dx <= _RMAX * _RMAX)
_K = len(_OFFS)
_NW = 32
_TPW = -(-_NT // _NW)
_NEG = -1e30


def _resize_mat(Hi, Ho):
    import numpy as np
    ys = np.maximum((np.arange(Ho, dtype=np.float64) + 0.5) * (Hi / Ho) - 0.5,
                    0.0)
    y0 = np.clip(np.floor(ys).astype(np.int64), 0, Hi - 1)
    y1 = np.minimum(y0 + 1, Hi - 1)
    wy = (ys - y0).astype(np.float32)
    W = np.zeros((Ho, Hi), np.float32)
    W[np.arange(Ho), y0] += 1.0 - wy
    W[np.arange(Ho), y1] += wy
    return jnp.asarray(W)


def _bilinear_resize(img, Ho, Wo):
    B, C, Hi, Wi = img.shape
    Wy = _resize_mat(Hi, Ho)
    Wx = _resize_mat(Wi, Wo)
    return jnp.einsum("oh,bchw,pw->bcop", Wy, img, Wx)


def _splat_i32(v):
    return jnp.full((16,), v, jnp.int32)


def _jbu_sc_body(featv_h, pv_h, ght_h, r2_h, out_h,
                 featv, pv, ght, r2v, wbuf, mbuf, dbuf, acc):
    wid = lax.axis_index("s") * 2 + lax.axis_index("c")
    pltpu.sync_copy(featv_h, featv)
    pltpu.sync_copy(pv_h, pv)
    col = [_splat_i32(c) for c in range(8)]
    xio = lax.iota(jnp.int32, 16).astype(jnp.float32)

    def do_tile(tile):
        i = tile // _WL
        j = tile - i * _WL
        pltpu.sync_copy(ght_h.at[tile], ght)
        pltpu.sync_copy(r2_h.at[tile], r2v)
        yb = (i * 16).astype(jnp.float32)
        xb = (j * 16).astype(jnp.float32)

        for k, (dy, dx) in enumerate(_OFFS):
            ni = jnp.clip(i + dy, 0, _HL - 1)
            nj = jnp.clip(j + dx, 0, _WL - 1)
            f = ni * _WL + nj
            fsp = _splat_i32(f)
            a = plsc.load_gather(pv, [fsp, col[0]])
            b = plsc.load_gather(pv, [fsp, col[1]])
            ct = plsc.load_gather(pv, [fsp, col[2]])
            st = plsc.load_gather(pv, [fsp, col[3]])
            cr = plsc.load_gather(pv, [fsp, col[4]])
            g0 = plsc.load_gather(pv, [fsp, col[5]])
            g1 = plsc.load_gather(pv, [fsp, col[6]])
            g2 = plsc.load_gather(pv, [fsp, col[7]])
            cxs = (nj.astype(jnp.float32) + 0.5) * float(_SCALE) - 0.5
            cys = (ni.astype(jnp.float32) + 0.5) * float(_SCALE) - 0.5
            dxv = xio + (xb - cxs)
            rad2 = float(dy * dy + dx * dx)

            def row1(r, _, k=k, dxv=dxv, a=a, b=b, ct=ct, st=st, cr=cr,
                     g0=g0, g1=g1, g2=g2, cys=cys):
                p = r * 16
                dyv = yb + r.astype(jnp.float32) - cys
                xp = dxv * ct + dyv * st
                yp = dyv * ct - dxv * st
                gh0 = ght[pl.ds(p, 16)]
                gh1 = ght[pl.ds(256 + p, 16)]
                gh2 = ght[pl.ds(512 + p, 16)]
                e0 = gh0 - g0
                e1 = gh1 - g1
                e2 = gh2 - g2
                d2 = e0 * e0 + e1 * e1 + e2 * e2
                lw = -(xp * xp * a + yp * yp * b) - d2 * cr
                mask = r2v[pl.ds(p, 16)] >= rad2
                lwm = jnp.where(mask, lw, _NEG)
                wbuf[k, pl.ds(p, 16)] = lwm
                if k == 0:
                    mbuf[pl.ds(p, 16)] = lwm
                else:
                    mbuf[pl.ds(p, 16)] = jnp.maximum(mbuf[pl.ds(p, 16)], lwm)
                return 0

            lax.fori_loop(0, 16, row1, 0, unroll=False)

        def row2(r, _):
            p = r * 16
            m = mbuf[pl.ds(p, 16)]
            den = jnp.zeros((16,), jnp.float32)
            for k in range(_K):
                s = jnp.exp(wbuf[k, pl.ds(p, 16)] - m)
                wbuf[k, pl.ds(p, 16)] = s
                den = den + s
            dbuf[pl.ds(p, 16)] = 1.0 / den
            return 0

        lax.fori_loop(0, 16, row2, 0, unroll=False)

        def cblk(cb, _):
            cols = [_splat_i32(cb * 8 + co) for co in range(8)]

            def rblk(rb, _, cols=cols, cb=cb):
                p0 = rb * 64
                accs = [[jnp.zeros((16,), jnp.float32) for _ in range(4)]
                        for _ in range(8)]
                for k, (dy, dx) in enumerate(_OFFS):
                    ni = jnp.clip(i + dy, 0, _HL - 1)
                    nj = jnp.clip(j + dx, 0, _WL - 1)
                    fsp = _splat_i32(ni * _WL + nj)
                    sv = [wbuf[k, pl.ds(p0 + rr * 16, 16)] for rr in range(4)]
                    for co in range(8):
                        fv = plsc.load_gather(featv, [fsp, cols[co]])
                        for rr in range(4):
                            accs[co][rr] = accs[co][rr] + fv * sv[rr]
                for rr in range(4):
                    inv = dbuf[pl.ds(p0 + rr * 16, 16)]
                    for co in range(8):
                        acc[cb * 8 + co, pl.ds(p0 + rr * 16, 16)] = \
                            accs[co][rr] * inv
                return 0

            lax.fori_loop(0, 4, rblk, 0, unroll=False)
            return 0

        lax.fori_loop(0, 8, cblk, 0, unroll=False)
        pltpu.sync_copy(acc, out_h.at[tile])

    def tloop(t, _):
        tile = wid + _NW * t

        @pl.when(tile < _NT)
        def _():
            do_tile(tile)

        return 0

    lax.fori_loop(0, _TPW, tloop, 0, unroll=False)


@jax.jit
def kernel(feat_lr, guide_hr, sx_raw, sy_raw, th_raw, sr_raw):
    B, C, Hl, Wl = feat_lr.shape
    _, _, Hh, Wh = guide_hr.shape
    sigma_x = jnp.exp(sx_raw)
    sigma_y = jnp.exp(sy_raw)
    theta = math.pi * jnp.tanh(th_raw)
    sigma_r = jnp.exp(sr_raw)
    sx = jnp.maximum(sigma_x, 1e-6)[0, 0]
    sy = jnp.maximum(sigma_y, 1e-6)[0, 0]
    sr = jnp.maximum(sigma_r, 1e-6)[0, 0]
    a_m = 1.0 / (2.0 * sx * sx + 1e-8)
    b_m = 1.0 / (2.0 * sy * sy + 1e-8)
    cr_m = 1.0 / (2.0 * sr * sr + 1e-8)
    cos_m = jnp.cos(theta[0, 0])
    sin_m = jnp.sin(theta[0, 0])
    glr = _bilinear_resize(guide_hr, Hl, Wl)[0]
    pv = jnp.zeros((Hl, Wl, 16), jnp.float32)
    for cidx, m in enumerate([a_m, b_m, cos_m, sin_m, cr_m,
                              glr[0], glr[1], glr[2]]):
        pv = pv.at[:, :, cidx].set(m)
    pv = pv.reshape(_NT, 16)
    sigma_eff = jnp.maximum(sigma_x, sigma_y)
    sigma_eff_hr = _bilinear_resize(sigma_eff, Hh, Wh)[0, 0]
    R_map = jnp.clip(jnp.ceil(2.0 * sigma_eff_hr), 1, _RMAX)
    r2 = (R_map * R_map).astype(jnp.float32)
    r2t = r2.reshape(Hl, 16, Wl, 16).transpose(0, 2, 1, 3).reshape(_NT, 256)
    featv = feat_lr[0].transpose(1, 2, 0).reshape(_NT, _C)
    ght = (guide_hr[0].reshape(3, Hl, 16, Wl, 16)
           .transpose(1, 3, 0, 2, 4).reshape(_NT, 3 * 256))

    mesh = plsc.VectorSubcoreMesh(core_axis_name="c", subcore_axis_name="s",
                                  num_cores=2, num_subcores=16)
    out_t = pl.kernel(
        _jbu_sc_body,
        mesh=mesh,
        compiler_params=pltpu.CompilerParams(needs_layout_passes=False),
        out_type=jax.ShapeDtypeStruct((_NT, _C, 256), jnp.float32),
        scratch_types=[
            pltpu.VMEM((_NT, _C), jnp.float32),
            pltpu.VMEM((_NT, 16), jnp.float32),
            pltpu.VMEM((3 * 256,), jnp.float32),
            pltpu.VMEM((256,), jnp.float32),
            pltpu.VMEM((_K, 256), jnp.float32),
            pltpu.VMEM((256,), jnp.float32),
            pltpu.VMEM((256,), jnp.float32),
            pltpu.VMEM((_C, 256), jnp.float32),
        ],
    )(featv, pv, ght, r2t)
    out = (out_t.reshape(Hl, Wl, _C, 16, 16)
           .transpose(2, 0, 3, 1, 4).reshape(1, _C, Hh, Wh))
    return out


# speedup vs baseline: 713.1426x; 1.0025x over previous
import functools
import math

import jax
import jax.numpy as jnp
from jax import lax
from jax.experimental import pallas as pl
from jax.experimental.pallas import tpu as pltpu
from jax.experimental.pallas import tpu_sc as plsc

_SCALE = 16
_RMAX = 3
_HL = 14
_WL = 14
_C = 64
_NT = _HL * _WL
_OFFS = tuple((dy, dx) for dy in range(-_RMAX, _RMAX + 1)
              for dx in range(-_RMAX, _RMAX + 1)
              if dy * dy + dx * dx <= _RMAX * _RMAX)
_K = len(_OFFS)
_NW = 32
_TPW = -(-_NT // _NW)
_NEG = -1e30


def _resize_mat(Hi, Ho):
    import numpy as np
    ys = np.maximum((np.arange(Ho, dtype=np.float64) + 0.5) * (Hi / Ho) - 0.5,
                    0.0)
    y0 = np.clip(np.floor(ys).astype(np.int64), 0, Hi - 1)
    y1 = np.minimum(y0 + 1, Hi - 1)
    wy = (ys - y0).astype(np.float32)
    W = np.zeros((Ho, Hi), np.float32)
    W[np.arange(Ho), y0] += 1.0 - wy
    W[np.arange(Ho), y1] += wy
    return jnp.asarray(W)


def _bilinear_resize(img, Ho, Wo):
    B, C, Hi, Wi = img.shape
    Wy = _resize_mat(Hi, Ho)
    Wx = _resize_mat(Wi, Wo)
    return jnp.einsum("oh,bchw,pw->bcop", Wy, img, Wx,
                      precision=jax.lax.Precision.HIGHEST)


def _splat_i32(v):
    return jnp.full((16,), v, jnp.int32)


def _jbu_sc_body(featv_h, pv_h, ght_h, r2_h, out_h,
                 featv, pv, ght, r2v, wbuf, mbuf, dbuf, acc):
    wid = lax.axis_index("s") * 2 + lax.axis_index("c")
    pltpu.sync_copy(featv_h, featv)
    pltpu.sync_copy(pv_h, pv)
    col = [_splat_i32(c) for c in range(8)]
    xio = lax.iota(jnp.int32, 16).astype(jnp.float32)

    def do_tile(tile):
        i = tile // _WL
        j = tile - i * _WL
        pltpu.sync_copy(ght_h.at[tile], ght)
        pltpu.sync_copy(r2_h.at[tile], r2v)
        yb = (i * 16).astype(jnp.float32)
        xb = (j * 16).astype(jnp.float32)

        for k, (dy, dx) in enumerate(_OFFS):
            ni = jnp.clip(i + dy, 0, _HL - 1)
            nj = jnp.clip(j + dx, 0, _WL - 1)
            f = ni * _WL + nj
            fsp = _splat_i32(f)
            a = plsc.load_gather(pv, [fsp, col[0]])
            b = plsc.load_gather(pv, [fsp, col[1]])
            ct = plsc.load_gather(pv, [fsp, col[2]])
            st = plsc.load_gather(pv, [fsp, col[3]])
            cr = plsc.load_gather(pv, [fsp, col[4]])
            g0 = plsc.load_gather(pv, [fsp, col[5]])
            g1 = plsc.load_gather(pv, [fsp, col[6]])
            g2 = plsc.load_gather(pv, [fsp, col[7]])
            cxs = (nj.astype(jnp.float32) + 0.5) * float(_SCALE) - 0.5
            cys = (ni.astype(jnp.float32) + 0.5) * float(_SCALE) - 0.5
            dxv = xio + (xb - cxs)
            rad2 = float(dy * dy + dx * dx)

            def row1(r, _, k=k, dxv=dxv, a=a, b=b, ct=ct, st=st, cr=cr,
                     g0=g0, g1=g1, g2=g2, cys=cys):
                p = r * 16
                dyv = yb + r.astype(jnp.float32) - cys
                xp = dxv * ct + dyv * st
                yp = dyv * ct - dxv * st
                gh0 = ght[pl.ds(p, 16)]
                gh1 = ght[pl.ds(256 + p, 16)]
                gh2 = ght[pl.ds(512 + p, 16)]
                e0 = gh0 - g0
                e1 = gh1 - g1
                e2 = gh2 - g2
                d2 = e0 * e0 + e1 * e1 + e2 * e2
                lw = -(xp * xp * a + yp * yp * b) - d2 * cr
                mask = r2v[pl.ds(p, 16)] >= rad2
                lwm = jnp.where(mask, lw, _NEG)
                wbuf[k, pl.ds(p, 16)] = lwm
                if k == 0:
                    mbuf[pl.ds(p, 16)] = lwm
                else:
                    mbuf[pl.ds(p, 16)] = jnp.maximum(mbuf[pl.ds(p, 16)], lwm)
                return 0

            lax.fori_loop(0, 16, row1, 0, unroll=False)

        def row2(r, _):
            p = r * 16
            m = mbuf[pl.ds(p, 16)]
            den = jnp.zeros((16,), jnp.float32)
            for k in range(_K):
                s = jnp.exp(wbuf[k, pl.ds(p, 16)] - m)
                wbuf[k, pl.ds(p, 16)] = s
                den = den + s
            dbuf[pl.ds(p, 16)] = 1.0 / den
            return 0

        lax.fori_loop(0, 16, row2, 0, unroll=False)

        def cblk(cb, _):
            cols = [_splat_i32(cb * 8 + co) for co in range(8)]

            def rblk(rb, _, cols=cols, cb=cb):
                p0 = rb * 64
                accs = [[jnp.zeros((16,), jnp.float32) for _ in range(4)]
                        for _ in range(8)]
                for k, (dy, dx) in enumerate(_OFFS):
                    ni = jnp.clip(i + dy, 0, _HL - 1)
                    nj = jnp.clip(j + dx, 0, _WL - 1)
                    fsp = _splat_i32(ni * _WL + nj)
                    sv = [wbuf[k, pl.ds(p0 + rr * 16, 16)] for rr in range(4)]
                    for co in range(8):
                        fv = plsc.load_gather(featv, [fsp, cols[co]])
                        for rr in range(4):
                            accs[co][rr] = accs[co][rr] + fv * sv[rr]
                for rr in range(4):
                    inv = dbuf[pl.ds(p0 + rr * 16, 16)]
                    for co in range(8):
                        acc[cb * 8 + co, pl.ds(p0 + rr * 16, 16)] = \
                            accs[co][rr] * inv
                return 0

            lax.fori_loop(0, 4, rblk, 0, unroll=False)
            return 0

        lax.fori_loop(0, 8, cblk, 0, unroll=False)
        pltpu.sync_copy(acc, out_h.at[tile])

    def tloop(t, _):
        tile = wid + _NW * t

        @pl.when(tile < _NT)
        def _():
            do_tile(tile)

        return 0

    lax.fori_loop(0, _TPW, tloop, 0, unroll=False)


@jax.jit
def kernel(feat_lr, guide_hr, sx_raw, sy_raw, th_raw, sr_raw):
    B, C, Hl, Wl = feat_lr.shape
    _, _, Hh, Wh = guide_hr.shape
    sigma_x = jnp.exp(sx_raw)
    sigma_y = jnp.exp(sy_raw)
    theta = math.pi * jnp.tanh(th_raw)
    sigma_r = jnp.exp(sr_raw)
    sx = jnp.maximum(sigma_x, 1e-6)[0, 0]
    sy = jnp.maximum(sigma_y, 1e-6)[0, 0]
    sr = jnp.maximum(sigma_r, 1e-6)[0, 0]
    a_m = 1.0 / (2.0 * sx * sx + 1e-8)
    b_m = 1.0 / (2.0 * sy * sy + 1e-8)
    cr_m = 1.0 / (2.0 * sr * sr + 1e-8)
    cos_m = jnp.cos(theta[0, 0])
    sin_m = jnp.sin(theta[0, 0])
    glr = _bilinear_resize(guide_hr, Hl, Wl)[0]
    pv = jnp.zeros((Hl, Wl, 16), jnp.float32)
    for cidx, m in enumerate([a_m, b_m, cos_m, sin_m, cr_m,
                              glr[0], glr[1], glr[2]]):
        pv = pv.at[:, :, cidx].set(m)
    pv = pv.reshape(_NT, 16)
    sigma_eff = jnp.maximum(sigma_x, sigma_y)
    sigma_eff_hr = _bilinear_resize(sigma_eff, Hh, Wh)[0, 0]
    R_map = jnp.clip(jnp.ceil(2.0 * sigma_eff_hr), 1, _RMAX)
    r2 = (R_map * R_map).astype(jnp.float32)
    r2t = r2.reshape(Hl, 16, Wl, 16).transpose(0, 2, 1, 3).reshape(_NT, 256)
    featv = feat_lr[0].transpose(1, 2, 0).reshape(_NT, _C)
    ght = (guide_hr[0].reshape(3, Hl, 16, Wl, 16)
           .transpose(1, 3, 0, 2, 4).reshape(_NT, 3 * 256))

    mesh = plsc.VectorSubcoreMesh(core_axis_name="c", subcore_axis_name="s",
                                  num_cores=2, num_subcores=16)
    out_t = pl.kernel(
        _jbu_sc_body,
        mesh=mesh,
        compiler_params=pltpu.CompilerParams(needs_layout_passes=False),
        out_type=jax.ShapeDtypeStruct((_NT, _C, 256), jnp.float32),
        scratch_types=[
            pltpu.VMEM((_NT, _C), jnp.float32),
            pltpu.VMEM((_NT, 16), jnp.float32),
            pltpu.VMEM((3 * 256,), jnp.float32),
            pltpu.VMEM((256,), jnp.float32),
            pltpu.VMEM((_K, 256), jnp.float32),
            pltpu.VMEM((256,), jnp.float32),
            pltpu.VMEM((256,), jnp.float32),
            pltpu.VMEM((_C, 256), jnp.float32),
        ],
    )(featv, pv, ght, r2t)
    out = (out_t.reshape(Hl, Wl, _C, 16, 16)
           .transpose(2, 0, 3, 1, 4).reshape(1, _C, Hh, Wh))
    return out
